# Initial kernel scaffold; baseline (speedup 1.0000x reference)
import jax
import jax.numpy as jnp
from jax.experimental import pallas as pl


def kernel(x, edge_index, edge_weight, W_enc, b_enc, Wh, W_dec):
    raise NotImplementedError("write your pallas kernel here")


# speedup vs baseline: 2.0344x; 2.0344x over previous
import dataclasses
import functools

import jax
import jax.numpy as jnp
from jax import lax
from jax.experimental import pallas as pl
from jax.experimental.pallas import tpu as pltpu
from jax.experimental.pallas import tpu_sc as plsc

N = 10000
E = 320000
D = 128
ALPHA = 1.0
GAMMA = 0.9
MAX_ITER = 10

NC = 2
NS = 16
NW = NC * NS
CPW = D // NW
LANES = 16
CHUNK = 8000
NCHUNK = E // CHUNK
GROUPS = CHUNK // LANES

@functools.cache
def _make_spmm_sc():
    mesh = plsc.VectorSubcoreMesh(core_axis_name="c", subcore_axis_name="s")
    cp = pltpu.CompilerParams()
    if "needs_layout_passes" in pltpu.CompilerParams.__dataclass_fields__:
        cp = dataclasses.replace(cp, needs_layout_passes=False)
    return pl.kernel(
        _spmm_body,
        out_type=jax.ShapeDtypeStruct((D, N), jnp.float32),
        mesh=mesh,
        compiler_params=cp,
        scratch_types=[
            pltpu.VMEM((CPW, N), jnp.float32),
            pltpu.VMEM((CPW, N), jnp.float32),
            pltpu.VMEM((CHUNK,), jnp.int32),
            pltpu.VMEM((CHUNK,), jnp.float32),
        ],
    )


def _spmm_body(v_hbm, e_hbm, w_hbm, out_hbm, v_loc, acc, ebuf, wbuf):
    wid = lax.axis_index("s") * NC + lax.axis_index("c")
    row0 = wid * CPW
    pltpu.sync_copy(v_hbm.at[pl.ds(row0, CPW)], v_loc)

    zero16 = jnp.zeros((LANES,), jnp.float32)
    for c in range(CPW):
        @pl.loop(0, N // LANES)
        def _(i, c=c):
            acc[c, pl.ds(i * LANES, LANES)] = zero16

    cvecs = [jnp.full((LANES,), c, jnp.int32) for c in range(CPW)]

    @pl.loop(0, NCHUNK)
    def _(ci):
        pltpu.sync_copy(e_hbm.at[pl.ds(ci * CHUNK, CHUNK)], ebuf)
        pltpu.sync_copy(w_hbm.at[pl.ds(ci * CHUNK, CHUNK)], wbuf)

        @pl.loop(0, GROUPS)
        def _(g):
            e16 = ebuf[pl.ds(g * LANES, LANES)]
            w16 = wbuf[pl.ds(g * LANES, LANES)]
            srcv = lax.shift_right_logical(e16, 14)
            dstv = lax.bitwise_and(e16, jnp.int32((1 << 14) - 1))
            for c in range(CPW):
                gv = plsc.load_gather(v_loc, [cvecs[c], srcv])
                plsc.addupdate_scatter(acc, [cvecs[c], dstv], gv * w16)

    pltpu.sync_copy(acc, out_hbm.at[pl.ds(row0, CPW)])


def _enc_body(xt_ref, we_ref, be_ref, b_ref, v_ref):
    b_val = jnp.dot(we_ref[...], xt_ref[...],
                    preferred_element_type=jnp.float32) + be_ref[...]
    b_ref[...] = b_val
    v_ref[...] = ALPHA * b_val


def _upd_body(s_ref, v_ref, u_ref, b_ref, wh_ref, u2_ref, v2_ref, z_ref):
    m = jnp.dot(wh_ref[...], s_ref[...], preferred_element_type=jnp.float32)
    v = v_ref[...]
    u = u_ref[...]
    c = ALPHA / (1.0 + ALPHA)
    z_half = (v + c * GAMMA * m) / (1.0 + ALPHA)
    u_half = 2.0 * z_half - u
    z = jnp.maximum(u_half, 0.0)
    u2 = 2.0 * z - u_half
    u2_ref[...] = u2
    v2_ref[...] = u2 + ALPHA * b_ref[...]
    z_ref[...] = z


def _dec_body(z_ref, wd_ref, o_ref):
    o_ref[...] = jnp.dot(wd_ref[...], jnp.maximum(z_ref[...], 0.0),
                         preferred_element_type=jnp.float32)


_F32 = functools.partial(jax.ShapeDtypeStruct, dtype=jnp.float32)

_enc = pl.pallas_call(
    _enc_body, out_shape=(_F32((D, N)), _F32((D, N))))
_upd = pl.pallas_call(
    _upd_body, out_shape=(_F32((D, N)), _F32((D, N)), _F32((D, N))))
_dec = pl.pallas_call(_dec_body, out_shape=_F32((D, N)))


def kernel(x, edge_index, edge_weight, W_enc, b_enc, Wh, W_dec):
    xt = jnp.swapaxes(x, 0, 1)
    packed = jnp.bitwise_or(
        jnp.left_shift(edge_index[0], 14), edge_index[1])
    b, v = _enc(xt, W_enc, b_enc.reshape(D, 1))
    u = jnp.zeros((D, N), jnp.float32)
    z = None
    spmm = _make_spmm_sc()
    for _ in range(MAX_ITER):
        s = spmm(v, packed, edge_weight)
        u, v, z = _upd(s, v, u, b, Wh)
    out_t = _dec(z, W_dec)
    return jnp.swapaxes(out_t, 0, 1)
